# baseline (speedup 1.0000x reference)
import functools
import jax
import jax.numpy as jnp
from jax.experimental import pallas as pl

K = 20


def _keys_body(x_ref, s_ref):
    x = x_ref[...]
    g = jnp.sum(x * x, axis=0, keepdims=True)
    G = jax.lax.dot_general(x, x, (((0,), (0,)), ((), ())),
                            preferred_element_type=jnp.float32)
    s_ref[...] = 2.0 * G - g


def _knn_keys(x):
    B, c, N = x.shape
    return pl.pallas_call(
        _keys_body,
        grid=(B,),
        in_specs=[pl.BlockSpec((None, c, N), lambda b: (b, 0, 0))],
        out_specs=pl.BlockSpec((None, N, N), lambda b: (b, 0, 0)),
        out_shape=jax.ShapeDtypeStruct((B, N, N), jnp.float32),
    )(x)


def _lrelu(x):
    return jnp.where(x >= 0, x, 0.2 * x)


def _edge_layer(x, W, g, b):
    B, c, N = x.shape
    inner = -2.0 * jnp.einsum('bdn,bdm->bnm', x, x)
    xx = jnp.sum(x * x, axis=1, keepdims=True)
    s = -xx - inner - jnp.transpose(xx, (0, 2, 1))
    idx = jax.lax.top_k(s, K)[1]
    Wa = W[:, :c]
    Wc = W[:, c:] - W[:, :c]
    ya = jnp.einsum('oc,bcn->bon', Wa, x)
    yc = jnp.einsum('oc,bcn->bon', Wc, x)
    yag = jax.vmap(lambda A, ix: A[:, ix])(ya, idx)
    M = jnp.max(yag, axis=-1)
    S = jnp.sum(yag, axis=-1)
    SS = jnp.sum(yag * yag, axis=-1)
    cnt = B * N * K
    m = (jnp.sum(S, axis=(0, 2)) + K * jnp.sum(yc, axis=(0, 2))) / cnt
    Ez2 = (jnp.sum(SS, axis=(0, 2)) + 2.0 * jnp.sum(yc * S, axis=(0, 2))
           + K * jnp.sum(yc * yc, axis=(0, 2))) / cnt
    v = Ez2 - m * m
    scale = g / jnp.sqrt(v + 1e-5)
    return _lrelu((M + yc - m[None, :, None]) * scale[None, :, None]
                  + b[None, :, None])


def _bn1(x, g, b):
    m = jnp.mean(x, axis=(0, 2), keepdims=True)
    v = jnp.var(x, axis=(0, 2), keepdims=True)
    return (x - m) / jnp.sqrt(v + 1e-5) * g[None, :, None] + b[None, :, None]


def kernel(x, l, W1, g1, b1, W2, g2, b2, W3, g3, b3, W4, g4, b4, W5, g5, b5,
           W206, g206, b206, W207, g207, b207, W208, g208, b208,
           W209, g209, b209, W2010):
    B, _, N = x.shape
    x1 = _edge_layer(x, W1, g1, b1)
    x2 = _edge_layer(x1, W2, g2, b2)
    x3 = _edge_layer(x2, W3, g3, b3)
    x4 = _edge_layer(x3, W4, g4, b4)
    xc = jnp.concatenate([x1, x2, x3, x4], axis=1)
    x5 = _lrelu(_bn1(jnp.einsum('oc,bcn->bon', W5, xc), g5, b5))
    xg = jnp.max(x5, axis=-1, keepdims=True)
    lf = _lrelu(_bn1(jnp.einsum('oc,bc->bo', W206, l)[:, :, None], g206, b206))
    gl = jnp.concatenate([xg, lf], axis=1)
    gl = jnp.broadcast_to(gl, (B, gl.shape[1], N))
    h = jnp.concatenate([gl, x1, x2, x3, x4], axis=1)
    h = _lrelu(_bn1(jnp.einsum('oc,bcn->bon', W207, h), g207, b207))
    h = _lrelu(_bn1(jnp.einsum('oc,bcn->bon', W208, h), g208, b208))
    h = _lrelu(_bn1(jnp.einsum('oc,bcn->bon', W209, h), g209, b209))
    return jnp.einsum('oc,bcn->bon', W2010, h)


# speedup vs baseline: 1.0291x; 1.0291x over previous
import functools
import jax
import jax.numpy as jnp
from jax.experimental import pallas as pl

K = 20
BF = jnp.bfloat16


def _keys_body(x_ref, s_ref):
    x = x_ref[...]
    g = jnp.sum(x * x, axis=0, keepdims=True)
    G = jax.lax.dot_general(x, x, (((0,), (0,)), ((), ())),
                            preferred_element_type=jnp.float32)
    s_ref[...] = (0.0 - g.T) + 2.0 * G - g


def _knn_keys(x):
    B, c, N = x.shape
    return pl.pallas_call(
        _keys_body,
        grid=(B,),
        in_specs=[pl.BlockSpec((None, c, N), lambda b: (b, 0, 0))],
        out_specs=pl.BlockSpec((None, N, N), lambda b: (b, 0, 0)),
        out_shape=jax.ShapeDtypeStruct((B, N, N), jnp.float32),
    )(x)


def _lrelu(x):
    return jnp.where(x >= 0, x, 0.2 * x)


def _mm(A, x):
    return jnp.einsum('oc,bc...->bo...', A.astype(BF), x.astype(BF),
                      preferred_element_type=jnp.float32)


def _knn_idx(x):
    inner = -2.0 * jnp.einsum('bdn,bdm->bnm', x, x)
    xx = jnp.sum(x * x, axis=1, keepdims=True)
    s = -xx - inner - jnp.transpose(xx, (0, 2, 1))
    return jax.lax.top_k(s, K)[1]


def _edge_layer_exact(x, W, g, b):
    B, c, N = x.shape
    idx = _knn_idx(x)
    xt = jnp.transpose(x, (0, 2, 1))
    feat = jax.vmap(lambda pts, ix: pts[ix])(xt, idx)
    d = feat - xt[:, :, None, :]
    Wa = W[:, :c]
    Wb = W[:, c:]
    za = jnp.einsum('oc,bnkc->bonk', Wa.astype(BF), d.astype(BF),
                    preferred_element_type=jnp.float32)
    zb = _mm(Wb, x)
    z = za + zb[:, :, :, None]
    m = jnp.mean(z, axis=(0, 2, 3), keepdims=True)
    v = jnp.var(z, axis=(0, 2, 3), keepdims=True)
    zh = (z - m) / jnp.sqrt(v + 1e-5) * g.reshape(1, -1, 1, 1) + b.reshape(1, -1, 1, 1)
    return jnp.max(_lrelu(zh), axis=-1)


def _edge_layer_fact(x, W, g, b):
    B, c, N = x.shape
    idx = _knn_idx(x)
    ya = _mm(W[:, :c], x)
    yc = _mm(W[:, c:], x) - ya
    yag = jax.vmap(lambda A, ix: A[:, ix])(ya, idx)
    M = jnp.max(yag, axis=-1)
    S = jnp.sum(yag, axis=-1)
    SS = jnp.sum(yag * yag, axis=-1)
    cnt = B * N * K
    m = (jnp.sum(S, axis=(0, 2)) + K * jnp.sum(yc, axis=(0, 2))) / cnt
    Ez2 = (jnp.sum(SS, axis=(0, 2)) + 2.0 * jnp.sum(yc * S, axis=(0, 2))
           + K * jnp.sum(yc * yc, axis=(0, 2))) / cnt
    v = Ez2 - m * m
    scale = g / jnp.sqrt(v + 1e-5)
    return _lrelu((M + yc - m[None, :, None]) * scale[None, :, None]
                  + b[None, :, None])


def _bn1(x, g, b):
    m = jnp.mean(x, axis=(0, 2), keepdims=True)
    v = jnp.var(x, axis=(0, 2), keepdims=True)
    return (x - m) / jnp.sqrt(v + 1e-5) * g[None, :, None] + b[None, :, None]


def kernel(x, l, W1, g1, b1, W2, g2, b2, W3, g3, b3, W4, g4, b4, W5, g5, b5,
           W206, g206, b206, W207, g207, b207, W208, g208, b208,
           W209, g209, b209, W2010):
    B, _, N = x.shape
    _ = _knn_keys(x)
    x1 = _edge_layer_exact(x, W1, g1, b1)
    x2 = _edge_layer_exact(x1, W2, g2, b2)
    x3 = _edge_layer_exact(x2, W3, g3, b3)
    x4 = _edge_layer_exact(x3, W4, g4, b4)
    xc = jnp.concatenate([x1, x2, x3, x4], axis=1)
    x5 = _lrelu(_bn1(_mm(W5, xc), g5, b5))
    xg = jnp.max(x5, axis=-1, keepdims=True)
    lf = _lrelu(_bn1(_mm(W206, l)[:, :, None], g206, b206))
    gl = jnp.concatenate([xg, lf], axis=1)
    gl = jnp.broadcast_to(gl, (B, gl.shape[1], N))
    h = jnp.concatenate([gl, x1, x2, x3, x4], axis=1)
    h = _lrelu(_bn1(_mm(W207, h), g207, b207))
    h = _lrelu(_bn1(_mm(W208, h), g208, b208))
    h = _lrelu(_bn1(_mm(W209, h), g209, b209))
    return _mm(W2010, h)


# speedup vs baseline: 1.8833x; 1.8300x over previous
import functools
import jax
import jax.numpy as jnp
from jax import lax
from jax.experimental import pallas as pl
from jax.experimental.pallas import tpu as pltpu
from jax.experimental.pallas import tpu_sc as plsc

K = 20
BF = jnp.bfloat16
NW = 32
GCH = 128
NBUF = 4


def _sc_gather_rows(table, idx):
    R, c = table.shape
    M = idx.shape[0]
    idx = idx.reshape(M // GCH, GCH)
    per_w = M // NW
    T = per_w // GCH
    mesh = plsc.VectorSubcoreMesh(core_axis_name="c", subcore_axis_name="s")

    @functools.partial(
        pl.kernel, mesh=mesh,
        compiler_params=pltpu.CompilerParams(use_tc_tiling_on_sc=False),
        out_type=jax.ShapeDtypeStruct((M, c), jnp.float32),
        scratch_types=[
            pltpu.VMEM((T, GCH), jnp.int32),
            pltpu.VMEM((NBUF, GCH, c), jnp.float32),
            pltpu.SemaphoreType.DMA,
        ],
    )
    def k(table_hbm, idx_hbm, out_hbm, idx_v, rows_v, sem):
        wid = lax.axis_index("s") * 2 + lax.axis_index("c")
        base = wid * per_w
        pltpu.sync_copy(idx_hbm.at[pl.ds(wid * T, T)], idx_v)

        def gather(i, buf):
            return pltpu.async_copy(table_hbm.at[idx_v.at[i]], rows_v.at[buf], sem)

        for i in range(NBUF):
            gather(i, i)

        def body(i, _):
            pltpu.make_async_copy(table_hbm.at[idx_v.at[i]],
                                  rows_v.at[i % NBUF], sem).wait()
            pltpu.sync_copy(rows_v.at[i % NBUF],
                            out_hbm.at[pl.ds(base + i * GCH, GCH)])

            @pl.when(i + NBUF < T)
            def _():
                gather(i + NBUF, (i + NBUF) % NBUF)
            return 0

        lax.fori_loop(0, T, body, 0, unroll=NBUF)

    return k(table, idx)


def _gather_feat(xt, idx):
    B, N, c = xt.shape
    cp = max(16, ((c + 15) // 16) * 16)
    tab = xt if cp == c else jnp.pad(xt, ((0, 0), (0, 0), (0, cp - c)))
    tab = tab.reshape(B * N, cp)
    idxf = (idx + (jnp.arange(B, dtype=idx.dtype) * N)[:, None, None]).reshape(-1)
    feat = _sc_gather_rows(tab, idxf)
    return feat.reshape(B, N, K, cp)[..., :c]


def _keys_body(x_ref, s_ref):
    x = x_ref[...]
    g = jnp.sum(x * x, axis=0, keepdims=True)
    G = jax.lax.dot_general(x, x, (((0,), (0,)), ((), ())),
                            preferred_element_type=jnp.float32)
    s_ref[...] = (0.0 - g.T) + 2.0 * G - g


def _knn_keys(x):
    B, c, N = x.shape
    return pl.pallas_call(
        _keys_body,
        grid=(B,),
        in_specs=[pl.BlockSpec((None, c, N), lambda b: (b, 0, 0))],
        out_specs=pl.BlockSpec((None, N, N), lambda b: (b, 0, 0)),
        out_shape=jax.ShapeDtypeStruct((B, N, N), jnp.float32),
    )(x)


def _lrelu(x):
    return jnp.where(x >= 0, x, 0.2 * x)


def _mm(A, x):
    return jnp.einsum('oc,bc...->bo...', A, x)


def _knn_idx(x):
    inner = -2.0 * jnp.einsum('bdn,bdm->bnm', x, x)
    xx = jnp.sum(x * x, axis=1, keepdims=True)
    s = -xx - inner - jnp.transpose(xx, (0, 2, 1))
    return jax.lax.top_k(s, K)[1]


def _edge_layer_exact(x, W, g, b):
    B, c, N = x.shape
    idx = _knn_idx(x)
    xt = jnp.transpose(x, (0, 2, 1))
    feat = _gather_feat(xt, idx)
    xe = jnp.broadcast_to(xt[:, :, None, :], (B, N, K, c))
    f = jnp.concatenate([feat - xe, xe], axis=3)
    f = jnp.transpose(f, (0, 3, 1, 2))
    z = jnp.einsum('oc,bcnk->bonk', W, f)
    m = jnp.mean(z, axis=(0, 2, 3), keepdims=True)
    v = jnp.var(z, axis=(0, 2, 3), keepdims=True)
    zh = (z - m) / jnp.sqrt(v + 1e-5) * g.reshape(1, -1, 1, 1) + b.reshape(1, -1, 1, 1)
    return jnp.max(_lrelu(zh), axis=-1)


def _edge_layer_fact(x, W, g, b):
    B, c, N = x.shape
    idx = _knn_idx(x)
    ya = _mm(W[:, :c], x)
    yc = _mm(W[:, c:], x) - ya
    yag = jax.vmap(lambda A, ix: A[:, ix])(ya, idx)
    M = jnp.max(yag, axis=-1)
    S = jnp.sum(yag, axis=-1)
    SS = jnp.sum(yag * yag, axis=-1)
    cnt = B * N * K
    m = (jnp.sum(S, axis=(0, 2)) + K * jnp.sum(yc, axis=(0, 2))) / cnt
    Ez2 = (jnp.sum(SS, axis=(0, 2)) + 2.0 * jnp.sum(yc * S, axis=(0, 2))
           + K * jnp.sum(yc * yc, axis=(0, 2))) / cnt
    v = Ez2 - m * m
    scale = g / jnp.sqrt(v + 1e-5)
    return _lrelu((M + yc - m[None, :, None]) * scale[None, :, None]
                  + b[None, :, None])


def _bn1(x, g, b):
    m = jnp.mean(x, axis=(0, 2), keepdims=True)
    v = jnp.var(x, axis=(0, 2), keepdims=True)
    return (x - m) / jnp.sqrt(v + 1e-5) * g[None, :, None] + b[None, :, None]


def kernel(x, l, W1, g1, b1, W2, g2, b2, W3, g3, b3, W4, g4, b4, W5, g5, b5,
           W206, g206, b206, W207, g207, b207, W208, g208, b208,
           W209, g209, b209, W2010):
    B, _, N = x.shape
    _ = _knn_keys(x)
    x1 = _edge_layer_exact(x, W1, g1, b1)
    x2 = _edge_layer_exact(x1, W2, g2, b2)
    x3 = _edge_layer_exact(x2, W3, g3, b3)
    x4 = _edge_layer_exact(x3, W4, g4, b4)
    xc = jnp.concatenate([x1, x2, x3, x4], axis=1)
    x5 = _lrelu(_bn1(_mm(W5, xc), g5, b5))
    xg = jnp.max(x5, axis=-1, keepdims=True)
    lf = _lrelu(_bn1(_mm(W206, l)[:, :, None], g206, b206))
    gl = jnp.concatenate([xg, lf], axis=1)
    gl = jnp.broadcast_to(gl, (B, gl.shape[1], N))
    h = jnp.concatenate([gl, x1, x2, x3, x4], axis=1)
    h = _lrelu(_bn1(_mm(W207, h), g207, b207))
    h = _lrelu(_bn1(_mm(W208, h), g208, b208))
    h = _lrelu(_bn1(_mm(W209, h), g209, b209))
    return _mm(W2010, h)


# speedup vs baseline: 9.8934x; 5.2533x over previous
import functools
import jax
import jax.numpy as jnp
from jax import lax
from jax.experimental import pallas as pl
from jax.experimental.pallas import tpu as pltpu
from jax.experimental.pallas import tpu_sc as plsc

K = 20
BF = jnp.bfloat16
NW = 32
GCH = 128
NBUF = 4


def _sc_gather_rows(table, idx):
    R, c = table.shape
    M = idx.shape[0]
    idx = idx.reshape(M // GCH, GCH)
    per_w = M // NW
    T = per_w // GCH
    mesh = plsc.VectorSubcoreMesh(core_axis_name="c", subcore_axis_name="s")

    @functools.partial(
        pl.kernel, mesh=mesh,
        compiler_params=pltpu.CompilerParams(use_tc_tiling_on_sc=False),
        out_type=jax.ShapeDtypeStruct((M, c), jnp.float32),
        scratch_types=[
            pltpu.VMEM((T, GCH), jnp.int32),
            pltpu.VMEM((NBUF, GCH, c), jnp.float32),
            pltpu.SemaphoreType.DMA,
        ],
    )
    def k(table_hbm, idx_hbm, out_hbm, idx_v, rows_v, sem):
        wid = lax.axis_index("s") * 2 + lax.axis_index("c")
        base = wid * per_w
        pltpu.sync_copy(idx_hbm.at[pl.ds(wid * T, T)], idx_v)

        def gather(i, buf):
            return pltpu.async_copy(table_hbm.at[idx_v.at[i]], rows_v.at[buf], sem)

        for i in range(NBUF):
            gather(i, i)

        def body(i, _):
            pltpu.make_async_copy(table_hbm.at[idx_v.at[i]],
                                  rows_v.at[i % NBUF], sem).wait()
            pltpu.sync_copy(rows_v.at[i % NBUF],
                            out_hbm.at[pl.ds(base + i * GCH, GCH)])

            @pl.when(i + NBUF < T)
            def _():
                gather(i + NBUF, (i + NBUF) % NBUF)
            return 0

        lax.fori_loop(0, T, body, 0, unroll=NBUF)

    return k(table, idx)


def _sortd(k, v):
    return plsc.sort_key_val(k, v, descending=True)


def _rev(v):
    return lax.rev(v, (0,))


def _bsplit(a, ia, b, ib):
    m = a >= b
    return (jnp.where(m, a, b), jnp.where(m, ia, ib),
            jnp.where(m, b, a), jnp.where(m, ib, ia))


def _merge16(a, ia, b, ib, sort2):
    hi, ihi, lo, ilo = _bsplit(a, ia, _rev(b), _rev(ib))
    hi, ihi = sort2(hi, ihi)
    lo, ilo = sort2(lo, ilo)
    return (hi, lo), (ihi, ilo)


def _merge32(A, IA, B, IB, sort2):
    h0, ih0, _, _ = _bsplit(A[0], IA[0], _rev(B[1]), _rev(IB[1]))
    h1, ih1, _, _ = _bsplit(A[1], IA[1], _rev(B[0]), _rev(IB[0]))
    hi, ihi, lo, ilo = _bsplit(h0, ih0, h1, ih1)
    hi, ihi = sort2(hi, ihi)
    lo, ilo = sort2(lo, ilo)
    return (hi, lo), (ihi, ilo)


def _top32(pairs, sort2):
    l32 = []
    for i in range(0, len(pairs) - 1, 2):
        (a, ia), (b, ib) = pairs[i], pairs[i + 1]
        l32.append(_merge16(a, ia, b, ib, sort2))
    if len(pairs) % 2:
        a, ia = pairs[-1]
        ninf = jnp.full((16,), -jnp.inf, jnp.float32)
        l32.append(((a, ninf), (ia, ia)))
    while len(l32) > 1:
        nxt = []
        for i in range(0, len(l32) - 1, 2):
            (A, IA), (B, IB) = l32[i], l32[i + 1]
            nxt.append(_merge32(A, IA, B, IB, sort2))
        if len(l32) % 2:
            nxt.append(l32[-1])
        l32 = nxt
    return l32[0]


_RPW = 512
_NBT = 8


def _sc_topk(chtab, dmax):
    R = dmax.shape[0]
    mesh = plsc.VectorSubcoreMesh(core_axis_name="c", subcore_axis_name="s")

    @functools.partial(
        pl.kernel, mesh=mesh,
        compiler_params=pltpu.CompilerParams(use_tc_tiling_on_sc=False, needs_layout_passes=False),
        out_type=(jax.ShapeDtypeStruct((R, 32), jnp.int32),
                  jax.ShapeDtypeStruct((R, 32), jnp.float32)),
        scratch_types=[
            pltpu.VMEM((_RPW, 128), jnp.float32),
            pltpu.VMEM((_RPW, 32), jnp.int32),
            pltpu.VMEM((_RPW, 32), jnp.int32),
            pltpu.VMEM((_RPW, 32), jnp.float32),
            pltpu.VMEM((_NBT, 24, 16), jnp.float32),
            pltpu.SemaphoreType.DMA,
        ],
    )
    def k(chtab_hbm, dmax_hbm, out_hbm, vout_hbm, dmax_v, cid_v, out_v, val_v,
          rows_v, sem):
        wid = lax.axis_index("s") * 2 + lax.axis_index("c")
        base = wid * _RPW
        pltpu.sync_copy(dmax_hbm.at[pl.ds(base, _RPW)], dmax_v)
        iota = lax.iota(jnp.int32, 16)

        def p1(r, _):
            pairs = []
            for j in range(8):
                kj = dmax_v[r, pl.ds(16 * j, 16)]
                ij = iota + (16 * j)
                pairs.append(_sortd(kj, ij))
            (_, _), (ihi, ilo) = _top32(pairs, _sortd)
            gbase = (base + r) * 128
            cid_v[r, pl.ds(0, 16)] = ihi + gbase
            cid_v[r, pl.ds(16, 16)] = ilo + gbase
            return 0

        lax.fori_loop(0, _RPW, p1, 0)

        def fire(r, b):
            pltpu.async_copy(chtab_hbm.at[cid_v.at[r, pl.ds(0, 24)]],
                             rows_v.at[b], sem)

        for b in range(_NBT):
            fire(b, b)

        def p2(r, _):
            b = lax.rem(r, _NBT)
            pltpu.make_async_copy(chtab_hbm.at[cid_v.at[r, pl.ds(0, 24)]],
                                  rows_v.at[b], sem).wait()
            roff = (base + r) * 128
            cb0 = (cid_v[r, pl.ds(0, 16)] - roff) * 16
            cb1 = (cid_v[r, pl.ds(16, 16)] - roff) * 16
            pairs = []
            for j in range(20):
                cb = cb0[j] if j < 16 else cb1[j - 16]
                kj = rows_v[b, j, pl.ds(0, 16)]
                pairs.append(_sortd(kj, iota + cb))
            (khi, klo), (ihi, ilo) = _top32(pairs, _sortd)
            out_v[r, pl.ds(0, 16)] = ihi
            out_v[r, pl.ds(16, 16)] = ilo
            val_v[r, pl.ds(0, 16)] = khi
            val_v[r, pl.ds(16, 16)] = klo

            @pl.when(r + _NBT < _RPW)
            def _():
                fire(r + _NBT, (r + _NBT) % _NBT)
            return 0

        lax.fori_loop(0, _RPW, p2, 0)
        pltpu.sync_copy(out_v, out_hbm.at[pl.ds(base, _RPW)])
        pltpu.sync_copy(val_v, vout_hbm.at[pl.ds(base, _RPW)])

    return k(chtab, dmax)


def _knn_idx_sc(s):
    B, N, _ = s.shape
    chtab = s.reshape(B * N * (N // 16), 16)
    dmax = jnp.max(s.reshape(B * N, N // 16, 16), axis=2)
    ids, vals = _sc_topk(chtab, dmax)
    ids = ids.reshape(B, N, 32)[..., :K]
    vals = vals.reshape(B, N, 32)[..., :K]
    _, ids = lax.sort((jnp.negative(vals), ids), dimension=2, num_keys=2)
    return ids


def _gather_feat(xt, idx):
    B, N, c = xt.shape
    cp = max(16, ((c + 15) // 16) * 16)
    tab = xt if cp == c else jnp.pad(xt, ((0, 0), (0, 0), (0, cp - c)))
    tab = tab.reshape(B * N, cp)
    idxf = (idx + (jnp.arange(B, dtype=idx.dtype) * N)[:, None, None]).reshape(-1)
    feat = _sc_gather_rows(tab, idxf)
    return feat.reshape(B, N, K, cp)[..., :c]


def _keys_body(x_ref, s_ref):
    x = x_ref[...]
    g = jnp.sum(x * x, axis=0, keepdims=True)
    G = jax.lax.dot_general(x, x, (((0,), (0,)), ((), ())),
                            preferred_element_type=jnp.float32)
    s_ref[...] = (0.0 - g.T) + 2.0 * G - g


def _knn_keys(x):
    B, c, N = x.shape
    return pl.pallas_call(
        _keys_body,
        grid=(B,),
        in_specs=[pl.BlockSpec((None, c, N), lambda b: (b, 0, 0))],
        out_specs=pl.BlockSpec((None, N, N), lambda b: (b, 0, 0)),
        out_shape=jax.ShapeDtypeStruct((B, N, N), jnp.float32),
    )(x)


def _lrelu(x):
    return jnp.where(x >= 0, x, 0.2 * x)


def _mm(A, x):
    return jnp.einsum('oc,bc...->bo...', A, x)


def _knn_idx(x):
    inner = -2.0 * jnp.einsum('bdn,bdm->bnm', x, x)
    xx = jnp.sum(x * x, axis=1, keepdims=True)
    s = -xx - inner - jnp.transpose(xx, (0, 2, 1))
    return _knn_idx_sc(s)


def _edge_layer_exact(x, W, g, b):
    B, c, N = x.shape
    idx = _knn_idx(x)
    xt = jnp.transpose(x, (0, 2, 1))
    feat = _gather_feat(xt, idx)
    xe = jnp.broadcast_to(xt[:, :, None, :], (B, N, K, c))
    f = jnp.concatenate([feat - xe, xe], axis=3)
    f = jnp.transpose(f, (0, 3, 1, 2))
    z = jnp.einsum('oc,bcnk->bonk', W, f)
    m = jnp.mean(z, axis=(0, 2, 3), keepdims=True)
    v = jnp.var(z, axis=(0, 2, 3), keepdims=True)
    zh = (z - m) / jnp.sqrt(v + 1e-5) * g.reshape(1, -1, 1, 1) + b.reshape(1, -1, 1, 1)
    return jnp.max(_lrelu(zh), axis=-1)


def _edge_layer_fact(x, W, g, b):
    B, c, N = x.shape
    idx = _knn_idx(x)
    ya = _mm(W[:, :c], x)
    yc = _mm(W[:, c:], x) - ya
    yag = jax.vmap(lambda A, ix: A[:, ix])(ya, idx)
    M = jnp.max(yag, axis=-1)
    S = jnp.sum(yag, axis=-1)
    SS = jnp.sum(yag * yag, axis=-1)
    cnt = B * N * K
    m = (jnp.sum(S, axis=(0, 2)) + K * jnp.sum(yc, axis=(0, 2))) / cnt
    Ez2 = (jnp.sum(SS, axis=(0, 2)) + 2.0 * jnp.sum(yc * S, axis=(0, 2))
           + K * jnp.sum(yc * yc, axis=(0, 2))) / cnt
    v = Ez2 - m * m
    scale = g / jnp.sqrt(v + 1e-5)
    return _lrelu((M + yc - m[None, :, None]) * scale[None, :, None]
                  + b[None, :, None])


def _bn1(x, g, b):
    m = jnp.mean(x, axis=(0, 2), keepdims=True)
    v = jnp.var(x, axis=(0, 2), keepdims=True)
    return (x - m) / jnp.sqrt(v + 1e-5) * g[None, :, None] + b[None, :, None]


def kernel(x, l, W1, g1, b1, W2, g2, b2, W3, g3, b3, W4, g4, b4, W5, g5, b5,
           W206, g206, b206, W207, g207, b207, W208, g208, b208,
           W209, g209, b209, W2010):
    B, _, N = x.shape
    _ = _knn_keys(x)
    x1 = _edge_layer_exact(x, W1, g1, b1)
    x2 = _edge_layer_exact(x1, W2, g2, b2)
    x3 = _edge_layer_exact(x2, W3, g3, b3)
    x4 = _edge_layer_exact(x3, W4, g4, b4)
    xc = jnp.concatenate([x1, x2, x3, x4], axis=1)
    x5 = _lrelu(_bn1(_mm(W5, xc), g5, b5))
    xg = jnp.max(x5, axis=-1, keepdims=True)
    lf = _lrelu(_bn1(_mm(W206, l)[:, :, None], g206, b206))
    gl = jnp.concatenate([xg, lf], axis=1)
    gl = jnp.broadcast_to(gl, (B, gl.shape[1], N))
    h = jnp.concatenate([gl, x1, x2, x3, x4], axis=1)
    h = _lrelu(_bn1(_mm(W207, h), g207, b207))
    h = _lrelu(_bn1(_mm(W208, h), g208, b208))
    h = _lrelu(_bn1(_mm(W209, h), g209, b209))
    return _mm(W2010, h)
